# Initial kernel scaffold; baseline (speedup 1.0000x reference)
import jax
import jax.numpy as jnp
from jax.experimental import pallas as pl


def kernel(input, table, weights):
    raise NotImplementedError("write your pallas kernel here")


# speedup vs baseline: 9.2939x; 9.2939x over previous
import functools

import jax
import jax.numpy as jnp
from jax import lax
from jax.experimental import pallas as pl
from jax.experimental.pallas import tpu as pltpu
from jax.experimental.pallas import tpu_sc as plsc

B = 4096
L = 50
H = 64
LANES = 16
HV = H // LANES

NC, NS = 2, 16
NW = NC * NS
BPW = B // NW

CB = 16
STEPS = BPW // CB
GCH = 100
NG = (CB * L) // GCH
NCHUNK = (BPW * L) // GCH
NB = 8


def _body(table_hbm, idx_hbm, w_hbm, out_hbm,
          idx_v, rows0, rows1, w_v, out_v, sem0, sem1):
    wid = lax.axis_index("c") * NS + lax.axis_index("s")
    row_base = wid * BPW

    pltpu.sync_copy(idx_hbm.at[wid], idx_v)
    pltpu.sync_copy(w_hbm, w_v)

    rows_bufs = (rows0, rows1)
    sems = (sem0, sem1)

    def fire(g):
        buf = rows_bufs[g % 2]
        sem = sems[g % 2]
        descs = []
        for j in range(NG):
            descs.append(pltpu.async_copy(
                table_hbm.at[idx_v.at[g * NG + j]],
                buf.at[pl.ds(j * GCH, GCH)],
                sem))
        return descs

    pending = {0: fire(0)}

    for g in range(STEPS):
        if g + 1 < STEPS:
            pending[g + 1] = fire(g + 1)
        for d in pending.pop(g):
            d.wait()
        rows = rows_bufs[g % 2]

        for bb in range(CB // NB):
            def step(l, accs, bb=bb, rows=rows):
                out = []
                ws = [w_v[l, pl.ds(h * LANES, LANES)] for h in range(HV)]
                for r in range(NB):
                    ridx = (bb * NB + r) * L + l
                    for h in range(HV):
                        out.append(accs[r * HV + h]
                                   + rows[ridx, pl.ds(h * LANES, LANES)] * ws[h])
                return tuple(out)

            zero = jnp.zeros((LANES,), jnp.float32)
            accs = lax.fori_loop(0, L, step, (zero,) * (NB * HV))
            for r in range(NB):
                for h in range(HV):
                    out_v[bb * NB + r, pl.ds(h * LANES, LANES)] = accs[r * HV + h]

        pltpu.sync_copy(out_v, out_hbm.at[pl.ds(row_base + g * CB, CB)])


@functools.partial(jax.jit, static_argnums=())
def _bow(table, idx, w):
    mesh = plsc.VectorSubcoreMesh(core_axis_name="c", subcore_axis_name="s",
                                  num_cores=NC, num_subcores=NS)
    return pl.kernel(
        _body,
        out_type=jax.ShapeDtypeStruct((B, H), jnp.float32),
        mesh=mesh,
        compiler_params=pltpu.CompilerParams(use_tc_tiling_on_sc=False),
        scratch_types=[
            pltpu.VMEM((NCHUNK, GCH), jnp.int32),
            pltpu.VMEM((NG * GCH, H), jnp.float32),
            pltpu.VMEM((NG * GCH, H), jnp.float32),
            pltpu.VMEM((L, H), jnp.float32),
            pltpu.VMEM((CB, H), jnp.float32),
            pltpu.SemaphoreType.DMA,
            pltpu.SemaphoreType.DMA,
        ],
    )(table, idx, w)


def kernel(input, table, weights):
    idx = input.reshape(NW, NCHUNK, GCH)
    return _bow(table, idx, weights[:L])
